# baseline (speedup 1.0000x reference)
import functools

import jax
import jax.numpy as jnp
from jax import lax
from jax.experimental import pallas as pl
from jax.experimental.pallas import tpu as pltpu
from jax.experimental.pallas import tpu_sc as plsc

NC = 2
NS = 16
NW = NC * NS
SCH = 64
LANES = 16


def _round_up(v, m):
  return (v + m - 1) // m * m


def _make_seg_sum(nacc, nquads, feat, tc_tiling=False):
  rpt = nacc // NS
  nz = rpt // SCH

  mesh = plsc.VectorSubcoreMesh(core_axis_name="c", subcore_axis_name="s",
                                num_cores=NC, num_subcores=NS)

  out_type = jax.ShapeDtypeStruct((NC * nacc, feat), jnp.float32)
  scratch = [
      pltpu.VMEM((4, SCH), jnp.int32),
      pltpu.VMEM((4, SCH), jnp.int32),
      pltpu.VMEM((4, SCH), jnp.int32),
      pltpu.VMEM((4, SCH), jnp.int32),
      pltpu.VMEM((SCH,), jnp.int32),
      pltpu.VMEM((SCH, feat), jnp.float32),
      pltpu.VMEM((SCH, feat), jnp.float32),
      pltpu.VMEM((SCH, feat), jnp.float32),
      pltpu.VMEM((SCH, feat), jnp.float32),
      pltpu.VMEM_SHARED((nacc, feat), jnp.float32),
      pltpu.SemaphoreType.DMA,
      pltpu.SemaphoreType.DMA,
      pltpu.SemaphoreType.DMA,
      pltpu.SemaphoreType.DMA,
      pltpu.SemaphoreType.DMA,
      pltpu.SemaphoreType.DMA,
      pltpu.SemaphoreType.DMA,
      pltpu.SemaphoreType.DMA,
      pltpu.SemaphoreType.DMA,
      pltpu.SemaphoreType.DMA,
  ]

  def body(h_hbm, src_hbm, dst_hbm, out_hbm,
           qsrc0, qsrc1, qdst0, qdst1, dmy, b0, b1, b2, b3, acc,
           g0, g1, g2, g3, s0, s1, s2, s3, semsi, semdi):
    c = lax.axis_index("c")
    s = lax.axis_index("s")
    w = c * NS + s
    base = w * (nquads + 1)
    bufs = (b0, b1, b2, b3)
    gsem = (g0, g1, g2, g3)
    ssem = (s0, s1, s2, s3)
    qsrc = (qsrc0, qsrc1)
    qdst = (qdst0, qdst1)

    zero16 = jnp.zeros((LANES,), jnp.float32)

    def zrow(i, carry):
      r = i // (feat // LANES)
      k = i % (feat // LANES)
      b0[r, pl.ds(k * LANES, LANES)] = zero16
      return carry
    lax.fori_loop(0, SCH * (feat // LANES), zrow, 0)
    r0 = s * rpt
    for j in range(nz):
      pltpu.sync_copy(b0, acc.at[pl.ds(r0 + j * SCH, SCH)])

    iota16 = lax.iota(jnp.int32, LANES)
    ndmy = nacc - rpt
    del ndmy
    for k in range(SCH // LANES):
      dmy[pl.ds(k * LANES, LANES)] = iota16 + (nacc - SCH + k * LANES)

    pltpu.sync_copy(src_hbm.at[base], qsrc0)
    pltpu.sync_copy(dst_hbm.at[base], qdst0)

    plsc.subcore_barrier()

    for k in (2, 3):
      pltpu.async_copy(b0, acc.at[dmy], ssem[k], add=True)
    pltpu.async_copy(h_hbm.at[qsrc0.at[0]], b0, gsem[0])
    pltpu.async_copy(h_hbm.at[qsrc0.at[1]], b1, gsem[1])

    def octstep(g8, carry):
      for half in range(2):
        quad = g8 * 2 + half
        sq_cur, dq_cur = qsrc[half], qdst[half]
        sq_nxt, dq_nxt = qsrc[1 - half], qdst[1 - half]
        ridx = base + quad + 1
        for q4 in range(4):
          if q4 == 0:
            pltpu.async_copy(src_hbm.at[ridx], sq_nxt, semsi)
            pltpu.async_copy(dst_hbm.at[ridx], dq_nxt, semdi)
          nb = (q4 + 2) % 4
          pltpu.make_async_copy(bufs[nb], acc.at[dmy], ssem[nb]).wait()
          if q4 == 2:
            pltpu.make_async_copy(src_hbm.at[ridx], sq_nxt, semsi).wait()
            pltpu.make_async_copy(dst_hbm.at[ridx], dq_nxt, semdi).wait()
          nrow = sq_cur.at[q4 + 2] if q4 < 2 else sq_nxt.at[q4 - 2]
          pltpu.async_copy(h_hbm.at[nrow], bufs[nb], gsem[nb])
          pltpu.make_async_copy(h_hbm.at[sq_cur.at[q4]],
                                bufs[q4], gsem[q4]).wait()
          pltpu.async_copy(bufs[q4], acc.at[dq_cur.at[q4]],
                           ssem[q4], add=True)
      return carry
    lax.fori_loop(0, nquads // 2, octstep, 0)

    pltpu.make_async_copy(h_hbm.at[qsrc0.at[0]], b0, gsem[0]).wait()
    pltpu.make_async_copy(h_hbm.at[qsrc0.at[1]], b1, gsem[1]).wait()
    for k in (2, 3):
      pltpu.make_async_copy(bufs[k], acc.at[dmy], ssem[k]).wait()

    plsc.subcore_barrier()

    pltpu.sync_copy(acc.at[pl.ds(r0, rpt)],
                    out_hbm.at[pl.ds(c * nacc + r0, rpt)])

  return pl.kernel(
      body,
      out_type=out_type,
      mesh=mesh,
      scratch_types=scratch,
      compiler_params=pltpu.CompilerParams(use_tc_tiling_on_sc=tc_tiling),
  )


def _lstm_body(xr, wl, bl, wa, outr):
  xb = xr[...]
  gates = jnp.dot(xb, wl[...], preferred_element_type=jnp.float32) + bl[...]
  hid = gates.shape[-1] // 4
  g = gates[:, 2 * hid:3 * hid]
  o = gates[:, 3 * hid:]
  c = jax.nn.sigmoid(gates[:, :hid]) * jnp.tanh(g)
  h = jax.nn.sigmoid(o) * jnp.tanh(c)
  sc = jnp.dot(h, wa[...], preferred_element_type=jnp.float32)
  sc = sc - jnp.max(sc, axis=-1, keepdims=True)
  e = jnp.exp(sc)
  attn = e / jnp.sum(e, axis=-1, keepdims=True)
  outr[...] = jnp.concatenate(
      [attn * h, jnp.ones((xb.shape[0], LANES), jnp.float32)], axis=1)


def _sage_body(a0r, a1r, lr, wl, bl, wr, outr):
  hid = wl.shape[0]
  a0 = a0r[...]
  a1 = a1r[...]
  hh = lr[:, :hid]
  agg = a0[:, :hid] + a1[:, :hid] + hh
  deg = a0[:, hid:hid + 1] + a1[:, hid:hid + 1] + 1.0
  pre = (jnp.dot(agg / deg, wl[...], preferred_element_type=jnp.float32)
         + bl[...]
         + jnp.dot(hh, wr[...], preferred_element_type=jnp.float32))
  outr[...] = jnp.maximum(pre, 0.0)


def _final_body(lr, h1r, b0r, b1r, a0r, a1r, wl, bl, wr,
                cw1, cb1, cw2, cb2, mw, mb, pw, pb, outr):
  hid = wl.shape[0]
  hh = h1r[...]
  agg = b0r[...] + b1r[...] + hh
  deg = a0r[:, hid:hid + 1] + a1r[:, hid:hid + 1] + 1.0
  pre = (jnp.dot(agg / deg, wl[...], preferred_element_type=jnp.float32)
         + bl[...]
         + jnp.dot(hh, wr[...], preferred_element_type=jnp.float32))
  h2 = jnp.maximum(pre, 0.0)
  x0 = jnp.concatenate([lr[:, :hid], hh, h2], axis=1)
  xc = x0
  for cw, cb in ((cw1, cb1), (cw2, cb2)):
    t = jnp.dot(xc, cw[...], preferred_element_type=jnp.float32)
    xc = x0 * t + cb[...] + xc
  deep = jnp.maximum(
      jnp.dot(x0, mw[...], preferred_element_type=jnp.float32) + mb[...], 0.0)
  fused = jnp.concatenate([xc, deep], axis=1)
  outr[...] = jnp.dot(fused, pw[...],
                      preferred_element_type=jnp.float32) + pb[...]


def _full(shape):
  return pl.BlockSpec(shape, lambda i: (0,) * len(shape))


def _rows(bl, ncols):
  return pl.BlockSpec((bl, ncols), lambda i: (i, 0))


def kernel(x, edge_index, W_lstm, b_lstm, W_att, W_l1, b_l1, W_r1,
           W_l2, b_l2, W_r2, cross_w1, cross_b1, cross_w2, cross_b2,
           mlp_W1, mlp_b1, pred_W, pred_b):
  n, feat = x.shape
  hid = W_att.shape[0]
  e = edge_index.shape[1]
  d = 3 * hid

  nacc = _round_up(n + 8, NS * 128)
  np_ = nacc
  br = 512
  nblk = np_ // br
  cpw = _round_up(e, NW * SCH) // (NW * SCH)
  cpw = _round_up(cpw, 8)
  nquads = cpw // 4
  e_proc = NW * cpw * SCH

  src = edge_index[0]
  dst = edge_index[1]
  pe = e_proc - e
  fill = jnp.arange(pe, dtype=jnp.int32)
  src_p = jnp.concatenate([src, (fill * 37) % n])
  dst_p = jnp.concatenate([dst, n + fill % (nacc - n)])
  src4 = src_p.reshape(NW, nquads, 4, SCH)
  dst4 = dst_p.reshape(NW, nquads, 4, SCH)
  dfill = jnp.arange(NW * 4 * SCH, dtype=jnp.int32)
  src4 = jnp.concatenate(
      [src4, ((dfill * 31) % n).reshape(NW, 1, 4, SCH)], axis=1)
  dst4 = jnp.concatenate(
      [dst4, (n + dfill % (nacc - n)).reshape(NW, 1, 4, SCH)], axis=1)
  srcp = src4.reshape(NW * (nquads + 1), 4, SCH)
  dstp = dst4.reshape(NW * (nquads + 1), 4, SCH)

  xp = jnp.pad(x, ((0, np_ - n), (0, 0)))
  bl2d = b_lstm.reshape(1, -1)
  b1 = b_l1.reshape(1, -1)
  b2 = b_l2.reshape(1, -1)
  cw1 = cross_w1.reshape(-1, 1)
  cb1 = cross_b1.reshape(1, -1)
  cw2 = cross_w2.reshape(-1, 1)
  cb2 = cross_b2.reshape(1, -1)
  mdim = mlp_W1.shape[1]
  mpad = 128 - mdim
  mw = jnp.pad(mlp_W1, ((0, 0), (0, mpad)))
  mb = jnp.pad(mlp_b1, (0, mpad)).reshape(1, -1)
  pw = jnp.concatenate(
      [pred_W[:d], pred_W[d:], jnp.zeros((mpad, 1), jnp.float32)], axis=0)
  pb = pred_b.reshape(1, 1)

  grid = (nblk,)
  faug = hid + LANES

  def _shift(k):
    return pl.BlockSpec((br, faug), lambda i, k=k: (i + k, 0))

  lstm = pl.pallas_call(
      _lstm_body,
      grid=grid,
      in_specs=[_rows(br, feat), _full((feat, 4 * hid)),
                _full((1, 4 * hid)), _full((hid, hid))],
      out_specs=_rows(br, faug),
      out_shape=jax.ShapeDtypeStruct((np_, faug), jnp.float32),
  )(xp, W_lstm, bl2d, W_att)

  seg_deg = _make_seg_sum(nacc, nquads, faug)
  agg_f = seg_deg(lstm, srcp, dstp)

  h1 = pl.pallas_call(
      _sage_body,
      grid=grid,
      in_specs=[_shift(0), _shift(nblk), _rows(br, faug),
                _full((hid, hid)), _full((1, hid)), _full((hid, hid))],
      out_specs=_rows(br, hid),
      out_shape=jax.ShapeDtypeStruct((np_, hid), jnp.float32),
  )(agg_f, agg_f, lstm, W_l1, b1, W_r1)

  seg2 = _make_seg_sum(nacc, nquads, hid, tc_tiling=True)
  agg2_f = seg2(h1, srcp, dstp)

  def _shift2(k):
    return pl.BlockSpec((br, hid), lambda i, k=k: (i + k, 0))

  out = pl.pallas_call(
      _final_body,
      grid=grid,
      in_specs=[_rows(br, faug), _rows(br, hid),
                _shift2(0), _shift2(nblk),
                _shift(0), _shift(nblk),
                _full((hid, hid)), _full((1, hid)), _full((hid, hid)),
                _full((d, 1)), _full((1, d)), _full((d, 1)), _full((1, d)),
                _full((d, 128)), _full((1, 128)), _full((d + 128, 1)),
                _full((1, 1))],
      out_specs=_rows(br, 1),
      out_shape=jax.ShapeDtypeStruct((np_, 1), jnp.float32),
  )(lstm, h1, agg2_f, agg2_f, agg_f, agg_f, W_l2, b2, W_r2,
    cw1, cb1, cw2, cb2, mw, mb, pw, pb)

  return out[:n]


# speedup vs baseline: 1.0065x; 1.0065x over previous
import functools

import jax
import jax.numpy as jnp
from jax import lax
from jax.experimental import pallas as pl
from jax.experimental.pallas import tpu as pltpu
from jax.experimental.pallas import tpu_sc as plsc

NC = 2
NS = 16
NW = NC * NS
NB = 8
AH = 4
SCH = 32
LANES = 16


def _round_up(v, m):
  return (v + m - 1) // m * m


def _make_seg_sum(nacc, nquads, feat, tc_tiling=False):
  rpt = nacc // NS
  nz = rpt // SCH

  mesh = plsc.VectorSubcoreMesh(core_axis_name="c", subcore_axis_name="s",
                                num_cores=NC, num_subcores=NS)

  out_type = jax.ShapeDtypeStruct((NC * nacc, feat), jnp.float32)
  scratch = (
      [pltpu.VMEM((NB, SCH), jnp.int32) for _ in range(4)]
      + [pltpu.VMEM((SCH,), jnp.int32)]
      + [pltpu.VMEM((SCH, feat), jnp.float32) for _ in range(NB)]
      + [pltpu.VMEM_SHARED((nacc, feat), jnp.float32)]
      + [pltpu.SemaphoreType.DMA] * (2 * NB + 2)
  )

  def body(h_hbm, src_hbm, dst_hbm, out_hbm, *rest):
    qsrc0, qsrc1, qdst0, qdst1, dmy = rest[:5]
    bufs = rest[5:5 + NB]
    acc = rest[5 + NB]
    gsem = rest[6 + NB:6 + 2 * NB]
    ssem = rest[6 + 2 * NB:6 + 3 * NB]
    semsi, semdi = rest[6 + 3 * NB:]
    c = lax.axis_index("c")
    s = lax.axis_index("s")
    w = c * NS + s
    base = w * (nquads + 1)
    b0 = bufs[0]
    qsrc = (qsrc0, qsrc1)
    qdst = (qdst0, qdst1)

    zero16 = jnp.zeros((LANES,), jnp.float32)

    def zrow(i, carry):
      r = i // (feat // LANES)
      k = i % (feat // LANES)
      b0[r, pl.ds(k * LANES, LANES)] = zero16
      return carry
    lax.fori_loop(0, SCH * (feat // LANES), zrow, 0)
    r0 = s * rpt
    for j in range(nz):
      pltpu.sync_copy(b0, acc.at[pl.ds(r0 + j * SCH, SCH)])

    iota16 = lax.iota(jnp.int32, LANES)
    for k in range(SCH // LANES):
      dmy[pl.ds(k * LANES, LANES)] = iota16 + (nacc - SCH + k * LANES)

    pltpu.sync_copy(src_hbm.at[base], qsrc0)
    pltpu.sync_copy(dst_hbm.at[base], qdst0)

    plsc.subcore_barrier()

    for k in range(AH, NB):
      pltpu.async_copy(b0, acc.at[dmy], ssem[k], add=True)
    for k in range(AH):
      pltpu.async_copy(h_hbm.at[qsrc0.at[k]], bufs[k], gsem[k])

    def groupstep(gg, carry):
      for half in range(2):
        grp = gg * 2 + half
        sq_cur, dq_cur = qsrc[half], qdst[half]
        sq_nxt, dq_nxt = qsrc[1 - half], qdst[1 - half]
        ridx = base + grp + 1
        for q in range(NB):
          if q == 0:
            pltpu.async_copy(src_hbm.at[ridx], sq_nxt, semsi)
            pltpu.async_copy(dst_hbm.at[ridx], dq_nxt, semdi)
          nb = (q + AH) % NB
          pltpu.make_async_copy(bufs[nb], acc.at[dmy], ssem[nb]).wait()
          if q == NB - AH:
            pltpu.make_async_copy(src_hbm.at[ridx], sq_nxt, semsi).wait()
            pltpu.make_async_copy(dst_hbm.at[ridx], dq_nxt, semdi).wait()
          nrow = (sq_cur.at[q + AH] if q < NB - AH
                  else sq_nxt.at[q - (NB - AH)])
          pltpu.async_copy(h_hbm.at[nrow], bufs[nb], gsem[nb])
          pltpu.make_async_copy(h_hbm.at[sq_cur.at[q]],
                                bufs[q], gsem[q]).wait()
          pltpu.async_copy(bufs[q], acc.at[dq_cur.at[q]],
                           ssem[q], add=True)
      return carry
    lax.fori_loop(0, nquads // 2, groupstep, 0)

    for k in range(AH):
      pltpu.make_async_copy(h_hbm.at[qsrc0.at[k]], bufs[k], gsem[k]).wait()
    for k in range(AH, NB):
      pltpu.make_async_copy(bufs[k], acc.at[dmy], ssem[k]).wait()

    plsc.subcore_barrier()

    pltpu.sync_copy(acc.at[pl.ds(r0, rpt)],
                    out_hbm.at[pl.ds(c * nacc + r0, rpt)])

  return pl.kernel(
      body,
      out_type=out_type,
      mesh=mesh,
      scratch_types=scratch,
      compiler_params=pltpu.CompilerParams(use_tc_tiling_on_sc=tc_tiling),
  )


def _lstm_body(xr, wl, bl, wa, outr):
  xb = xr[...]
  gates = jnp.dot(xb, wl[...], preferred_element_type=jnp.float32) + bl[...]
  hid = gates.shape[-1] // 4
  g = gates[:, 2 * hid:3 * hid]
  o = gates[:, 3 * hid:]
  c = jax.nn.sigmoid(gates[:, :hid]) * jnp.tanh(g)
  h = jax.nn.sigmoid(o) * jnp.tanh(c)
  sc = jnp.dot(h, wa[...], preferred_element_type=jnp.float32)
  sc = sc - jnp.max(sc, axis=-1, keepdims=True)
  e = jnp.exp(sc)
  attn = e / jnp.sum(e, axis=-1, keepdims=True)
  outr[...] = jnp.concatenate(
      [attn * h, jnp.ones((xb.shape[0], LANES), jnp.float32)], axis=1)


def _sage_body(a0r, a1r, lr, wl, bl, wr, outr):
  hid = wl.shape[0]
  a0 = a0r[...]
  a1 = a1r[...]
  hh = lr[:, :hid]
  agg = a0[:, :hid] + a1[:, :hid] + hh
  deg = a0[:, hid:hid + 1] + a1[:, hid:hid + 1] + 1.0
  pre = (jnp.dot(agg / deg, wl[...], preferred_element_type=jnp.float32)
         + bl[...]
         + jnp.dot(hh, wr[...], preferred_element_type=jnp.float32))
  outr[...] = jnp.maximum(pre, 0.0)


def _final_body(lr, h1r, b0r, b1r, a0r, a1r, wl, bl, wr,
                cw1, cb1, cw2, cb2, mw, mb, pw, pb, outr):
  hid = wl.shape[0]
  hh = h1r[...]
  agg = b0r[...] + b1r[...] + hh
  deg = a0r[:, hid:hid + 1] + a1r[:, hid:hid + 1] + 1.0
  pre = (jnp.dot(agg / deg, wl[...], preferred_element_type=jnp.float32)
         + bl[...]
         + jnp.dot(hh, wr[...], preferred_element_type=jnp.float32))
  h2 = jnp.maximum(pre, 0.0)
  x0 = jnp.concatenate([lr[:, :hid], hh, h2], axis=1)
  xc = x0
  for cw, cb in ((cw1, cb1), (cw2, cb2)):
    t = jnp.dot(xc, cw[...], preferred_element_type=jnp.float32)
    xc = x0 * t + cb[...] + xc
  deep = jnp.maximum(
      jnp.dot(x0, mw[...], preferred_element_type=jnp.float32) + mb[...], 0.0)
  fused = jnp.concatenate([xc, deep], axis=1)
  outr[...] = jnp.dot(fused, pw[...],
                      preferred_element_type=jnp.float32) + pb[...]


def _full(shape):
  return pl.BlockSpec(shape, lambda i: (0,) * len(shape))


def _rows(bl, ncols):
  return pl.BlockSpec((bl, ncols), lambda i: (i, 0))


def kernel(x, edge_index, W_lstm, b_lstm, W_att, W_l1, b_l1, W_r1,
           W_l2, b_l2, W_r2, cross_w1, cross_b1, cross_w2, cross_b2,
           mlp_W1, mlp_b1, pred_W, pred_b):
  n, feat = x.shape
  hid = W_att.shape[0]
  e = edge_index.shape[1]
  d = 3 * hid

  nacc = _round_up(n + 8, NS * 128)
  np_ = nacc
  br = 512
  nblk = np_ // br
  cpw = _round_up(e, NW * SCH) // (NW * SCH)
  cpw = _round_up(cpw, 2 * NB)
  nquads = cpw // NB
  e_proc = NW * cpw * SCH

  src = edge_index[0]
  dst = edge_index[1]
  pe = e_proc - e
  fill = jnp.arange(pe, dtype=jnp.int32)
  src_p = jnp.concatenate([src, (fill * 37) % n])
  dst_p = jnp.concatenate([dst, n + fill % (nacc - n)])
  src4 = src_p.reshape(NW, nquads, NB, SCH)
  dst4 = dst_p.reshape(NW, nquads, NB, SCH)
  dfill = jnp.arange(NW * NB * SCH, dtype=jnp.int32)
  src4 = jnp.concatenate(
      [src4, ((dfill * 31) % n).reshape(NW, 1, NB, SCH)], axis=1)
  dst4 = jnp.concatenate(
      [dst4, (n + dfill % (nacc - n)).reshape(NW, 1, NB, SCH)], axis=1)
  srcp = src4.reshape(NW * (nquads + 1), NB, SCH)
  dstp = dst4.reshape(NW * (nquads + 1), NB, SCH)

  xp = jnp.pad(x, ((0, np_ - n), (0, 0)))
  bl2d = b_lstm.reshape(1, -1)
  b1 = b_l1.reshape(1, -1)
  b2 = b_l2.reshape(1, -1)
  cw1 = cross_w1.reshape(-1, 1)
  cb1 = cross_b1.reshape(1, -1)
  cw2 = cross_w2.reshape(-1, 1)
  cb2 = cross_b2.reshape(1, -1)
  mdim = mlp_W1.shape[1]
  mpad = 128 - mdim
  mw = jnp.pad(mlp_W1, ((0, 0), (0, mpad)))
  mb = jnp.pad(mlp_b1, (0, mpad)).reshape(1, -1)
  pw = jnp.concatenate(
      [pred_W[:d], pred_W[d:], jnp.zeros((mpad, 1), jnp.float32)], axis=0)
  pb = pred_b.reshape(1, 1)

  grid = (nblk,)
  faug = hid + LANES

  def _shift(k):
    return pl.BlockSpec((br, faug), lambda i, k=k: (i + k, 0))

  lstm = pl.pallas_call(
      _lstm_body,
      grid=grid,
      in_specs=[_rows(br, feat), _full((feat, 4 * hid)),
                _full((1, 4 * hid)), _full((hid, hid))],
      out_specs=_rows(br, faug),
      out_shape=jax.ShapeDtypeStruct((np_, faug), jnp.float32),
  )(xp, W_lstm, bl2d, W_att)

  seg_deg = _make_seg_sum(nacc, nquads, faug)
  agg_f = seg_deg(lstm, srcp, dstp)

  h1 = pl.pallas_call(
      _sage_body,
      grid=grid,
      in_specs=[_shift(0), _shift(nblk), _rows(br, faug),
                _full((hid, hid)), _full((1, hid)), _full((hid, hid))],
      out_specs=_rows(br, hid),
      out_shape=jax.ShapeDtypeStruct((np_, hid), jnp.float32),
  )(agg_f, agg_f, lstm, W_l1, b1, W_r1)

  seg2 = _make_seg_sum(nacc, nquads, hid)
  agg2_f = seg2(h1, srcp, dstp)

  def _shift2(k):
    return pl.BlockSpec((br, hid), lambda i, k=k: (i + k, 0))

  out = pl.pallas_call(
      _final_body,
      grid=grid,
      in_specs=[_rows(br, faug), _rows(br, hid),
                _shift2(0), _shift2(nblk),
                _shift(0), _shift(nblk),
                _full((hid, hid)), _full((1, hid)), _full((hid, hid)),
                _full((d, 1)), _full((1, d)), _full((d, 1)), _full((1, d)),
                _full((d, 128)), _full((1, 128)), _full((d + 128, 1)),
                _full((1, 1))],
      out_specs=_rows(br, 1),
      out_shape=jax.ShapeDtypeStruct((np_, 1), jnp.float32),
  )(lstm, h1, agg2_f, agg2_f, agg_f, agg_f, W_l2, b2, W_r2,
    cw1, cb1, cw2, cb2, mw, mb, pw, pb)

  return out[:n]


# speedup vs baseline: 1.0480x; 1.0412x over previous
import functools

import jax
import jax.numpy as jnp
from jax import lax
from jax.experimental import pallas as pl
from jax.experimental.pallas import tpu as pltpu
from jax.experimental.pallas import tpu_sc as plsc

NC = 2
NS = 16
NW = NC * NS
NB = 4
AH = 2
SCH = 64
LANES = 16


def _round_up(v, m):
  return (v + m - 1) // m * m


def _make_seg_sum(nacc, nquads, feat, tc_tiling=False):
  rpt = nacc // NS
  nz = rpt // SCH

  mesh = plsc.VectorSubcoreMesh(core_axis_name="c", subcore_axis_name="s",
                                num_cores=NC, num_subcores=NS)

  out_type = jax.ShapeDtypeStruct((NC * nacc, feat), jnp.float32)
  scratch = (
      [pltpu.VMEM((NB, SCH), jnp.int32) for _ in range(4)]
      + [pltpu.VMEM((SCH,), jnp.int32)]
      + [pltpu.VMEM((SCH, feat), jnp.float32) for _ in range(NB)]
      + [pltpu.VMEM_SHARED((nacc, feat), jnp.float32)]
      + [pltpu.SemaphoreType.DMA] * (2 * NB + 2)
  )

  def body(h_hbm, src_hbm, dst_hbm, out_hbm, *rest):
    qsrc0, qsrc1, qdst0, qdst1, dmy = rest[:5]
    bufs = rest[5:5 + NB]
    acc = rest[5 + NB]
    gsem = rest[6 + NB:6 + 2 * NB]
    ssem = rest[6 + 2 * NB:6 + 3 * NB]
    semsi, semdi = rest[6 + 3 * NB:]
    c = lax.axis_index("c")
    s = lax.axis_index("s")
    w = c * NS + s
    base = w * (nquads + 1)
    b0 = bufs[0]
    qsrc = (qsrc0, qsrc1)
    qdst = (qdst0, qdst1)

    zero16 = jnp.zeros((LANES,), jnp.float32)

    def zrow(i, carry):
      r = i // (feat // LANES)
      k = i % (feat // LANES)
      b0[r, pl.ds(k * LANES, LANES)] = zero16
      return carry
    lax.fori_loop(0, SCH * (feat // LANES), zrow, 0)
    r0 = s * rpt
    for j in range(nz):
      pltpu.sync_copy(b0, acc.at[pl.ds(r0 + j * SCH, SCH)])

    iota16 = lax.iota(jnp.int32, LANES)
    for k in range(SCH // LANES):
      dmy[pl.ds(k * LANES, LANES)] = iota16 + (nacc - SCH + k * LANES)

    pltpu.sync_copy(src_hbm.at[base], qsrc0)
    pltpu.sync_copy(dst_hbm.at[base], qdst0)

    plsc.subcore_barrier()

    for k in range(AH, NB):
      pltpu.async_copy(b0, acc.at[dmy], ssem[k], add=True)
    for k in range(AH):
      pltpu.async_copy(h_hbm.at[qsrc0.at[k]], bufs[k], gsem[k])

    def groupstep(gg, carry):
      for half in range(2):
        grp = gg * 2 + half
        sq_cur, dq_cur = qsrc[half], qdst[half]
        sq_nxt, dq_nxt = qsrc[1 - half], qdst[1 - half]
        ridx = base + grp + 1
        for q in range(NB):
          if q == 0:
            pltpu.async_copy(src_hbm.at[ridx], sq_nxt, semsi)
            pltpu.async_copy(dst_hbm.at[ridx], dq_nxt, semdi)
          nb = (q + AH) % NB
          pltpu.make_async_copy(bufs[nb], acc.at[dmy], ssem[nb]).wait()
          if q == NB - AH:
            pltpu.make_async_copy(src_hbm.at[ridx], sq_nxt, semsi).wait()
            pltpu.make_async_copy(dst_hbm.at[ridx], dq_nxt, semdi).wait()
          nrow = (sq_cur.at[q + AH] if q < NB - AH
                  else sq_nxt.at[q - (NB - AH)])
          pltpu.async_copy(h_hbm.at[nrow], bufs[nb], gsem[nb])
          pltpu.make_async_copy(h_hbm.at[sq_cur.at[q]],
                                bufs[q], gsem[q]).wait()
          pltpu.async_copy(bufs[q], acc.at[dq_cur.at[q]],
                           ssem[q], add=True)
      return carry
    lax.fori_loop(0, nquads // 2, groupstep, 0)

    for k in range(AH):
      pltpu.make_async_copy(h_hbm.at[qsrc0.at[k]], bufs[k], gsem[k]).wait()
    for k in range(AH, NB):
      pltpu.make_async_copy(bufs[k], acc.at[dmy], ssem[k]).wait()

    plsc.subcore_barrier()

    pltpu.sync_copy(acc.at[pl.ds(r0, rpt)],
                    out_hbm.at[pl.ds(c * nacc + r0, rpt)])

  return pl.kernel(
      body,
      out_type=out_type,
      mesh=mesh,
      scratch_types=scratch,
      compiler_params=pltpu.CompilerParams(use_tc_tiling_on_sc=tc_tiling),
  )


def _lstm_body(xr, wl, bl, wa, outr):
  xb = xr[...]
  gates = jnp.dot(xb, wl[...], preferred_element_type=jnp.float32) + bl[...]
  hid = gates.shape[-1] // 4
  g = gates[:, 2 * hid:3 * hid]
  o = gates[:, 3 * hid:]
  c = jax.nn.sigmoid(gates[:, :hid]) * jnp.tanh(g)
  h = jax.nn.sigmoid(o) * jnp.tanh(c)
  sc = jnp.dot(h, wa[...], preferred_element_type=jnp.float32)
  sc = sc - jnp.max(sc, axis=-1, keepdims=True)
  e = jnp.exp(sc)
  attn = e / jnp.sum(e, axis=-1, keepdims=True)
  outr[...] = jnp.concatenate(
      [attn * h, jnp.ones((xb.shape[0], LANES), jnp.float32)], axis=1)


def _sage_body(a0r, a1r, lr, wl, bl, wr, outr):
  hid = wl.shape[0]
  a0 = a0r[...]
  a1 = a1r[...]
  hh = lr[:, :hid]
  agg = a0[:, :hid] + a1[:, :hid] + hh
  deg = a0[:, hid:hid + 1] + a1[:, hid:hid + 1] + 1.0
  pre = (jnp.dot(agg / deg, wl[...], preferred_element_type=jnp.float32)
         + bl[...]
         + jnp.dot(hh, wr[...], preferred_element_type=jnp.float32))
  outr[...] = jnp.maximum(pre, 0.0)


def _final_body(lr, h1r, b0r, b1r, a0r, a1r, wl, bl, wr,
                cw1, cb1, cw2, cb2, mw, mb, pw, pb, outr):
  hid = wl.shape[0]
  hh = h1r[...]
  agg = b0r[...] + b1r[...] + hh
  deg = a0r[:, hid:hid + 1] + a1r[:, hid:hid + 1] + 1.0
  pre = (jnp.dot(agg / deg, wl[...], preferred_element_type=jnp.float32)
         + bl[...]
         + jnp.dot(hh, wr[...], preferred_element_type=jnp.float32))
  h2 = jnp.maximum(pre, 0.0)
  x0 = jnp.concatenate([lr[:, :hid], hh, h2], axis=1)
  xc = x0
  for cw, cb in ((cw1, cb1), (cw2, cb2)):
    t = jnp.dot(xc, cw[...], preferred_element_type=jnp.float32)
    xc = x0 * t + cb[...] + xc
  deep = jnp.maximum(
      jnp.dot(x0, mw[...], preferred_element_type=jnp.float32) + mb[...], 0.0)
  fused = jnp.concatenate([xc, deep], axis=1)
  outr[...] = jnp.dot(fused, pw[...],
                      preferred_element_type=jnp.float32) + pb[...]


def _full(shape):
  return pl.BlockSpec(shape, lambda i: (0,) * len(shape))


def _rows(bl, ncols):
  return pl.BlockSpec((bl, ncols), lambda i: (i, 0))


def kernel(x, edge_index, W_lstm, b_lstm, W_att, W_l1, b_l1, W_r1,
           W_l2, b_l2, W_r2, cross_w1, cross_b1, cross_w2, cross_b2,
           mlp_W1, mlp_b1, pred_W, pred_b):
  n, feat = x.shape
  hid = W_att.shape[0]
  e = edge_index.shape[1]
  d = 3 * hid

  nacc = _round_up(n + 8, NS * 128)
  np_ = nacc
  br = 1024
  nblk = np_ // br
  cpw = _round_up(e, NW * SCH) // (NW * SCH)
  cpw = _round_up(cpw, 2 * NB)
  nquads = cpw // NB
  e_proc = NW * cpw * SCH

  src = edge_index[0]
  dst = edge_index[1]
  pe = e_proc - e
  fill = jnp.arange(pe, dtype=jnp.int32)
  src_p = jnp.concatenate([src, (fill * 37) % n])
  dst_p = jnp.concatenate([dst, n + fill % (nacc - n)])
  src4 = src_p.reshape(NW, nquads, NB, SCH)
  dst4 = dst_p.reshape(NW, nquads, NB, SCH)
  dfill = jnp.arange(NW * NB * SCH, dtype=jnp.int32)
  src4 = jnp.concatenate(
      [src4, ((dfill * 31) % n).reshape(NW, 1, NB, SCH)], axis=1)
  dst4 = jnp.concatenate(
      [dst4, (n + dfill % (nacc - n)).reshape(NW, 1, NB, SCH)], axis=1)
  srcp = src4.reshape(NW * (nquads + 1), NB, SCH)
  dstp = dst4.reshape(NW * (nquads + 1), NB, SCH)

  xp = jnp.pad(x, ((0, np_ - n), (0, 0)))
  bl2d = b_lstm.reshape(1, -1)
  b1 = b_l1.reshape(1, -1)
  b2 = b_l2.reshape(1, -1)
  cw1 = cross_w1.reshape(-1, 1)
  cb1 = cross_b1.reshape(1, -1)
  cw2 = cross_w2.reshape(-1, 1)
  cb2 = cross_b2.reshape(1, -1)
  mdim = mlp_W1.shape[1]
  mpad = 128 - mdim
  mw = jnp.pad(mlp_W1, ((0, 0), (0, mpad)))
  mb = jnp.pad(mlp_b1, (0, mpad)).reshape(1, -1)
  pw = jnp.concatenate(
      [pred_W[:d], pred_W[d:], jnp.zeros((mpad, 1), jnp.float32)], axis=0)
  pb = pred_b.reshape(1, 1)

  grid = (nblk,)
  faug = hid + LANES

  def _shift(k):
    return pl.BlockSpec((br, faug), lambda i, k=k: (i + k, 0))

  lstm = pl.pallas_call(
      _lstm_body,
      grid=grid,
      in_specs=[_rows(br, feat), _full((feat, 4 * hid)),
                _full((1, 4 * hid)), _full((hid, hid))],
      out_specs=_rows(br, faug),
      out_shape=jax.ShapeDtypeStruct((np_, faug), jnp.float32),
  )(xp, W_lstm, bl2d, W_att)

  seg_deg = _make_seg_sum(nacc, nquads, faug)
  agg_f = seg_deg(lstm, srcp, dstp)

  h1 = pl.pallas_call(
      _sage_body,
      grid=grid,
      in_specs=[_shift(0), _shift(nblk), _rows(br, faug),
                _full((hid, hid)), _full((1, hid)), _full((hid, hid))],
      out_specs=_rows(br, hid),
      out_shape=jax.ShapeDtypeStruct((np_, hid), jnp.float32),
  )(agg_f, agg_f, lstm, W_l1, b1, W_r1)

  seg2 = _make_seg_sum(nacc, nquads, hid)
  agg2_f = seg2(h1, srcp, dstp)

  def _shift2(k):
    return pl.BlockSpec((br, hid), lambda i, k=k: (i + k, 0))

  out = pl.pallas_call(
      _final_body,
      grid=grid,
      in_specs=[_rows(br, faug), _rows(br, hid),
                _shift2(0), _shift2(nblk),
                _shift(0), _shift(nblk),
                _full((hid, hid)), _full((1, hid)), _full((hid, hid)),
                _full((d, 1)), _full((1, d)), _full((d, 1)), _full((1, d)),
                _full((d, 128)), _full((1, 128)), _full((d + 128, 1)),
                _full((1, 1))],
      out_specs=_rows(br, 1),
      out_shape=jax.ShapeDtypeStruct((np_, 1), jnp.float32),
  )(lstm, h1, agg2_f, agg2_f, agg_f, agg_f, W_l2, b2, W_r2,
    cw1, cb1, cw2, cb2, mw, mb, pw, pb)

  return out[:n]


# speedup vs baseline: 1.0569x; 1.0085x over previous
import functools

import jax
import jax.numpy as jnp
from jax import lax
from jax.experimental import pallas as pl
from jax.experimental.pallas import tpu as pltpu
from jax.experimental.pallas import tpu_sc as plsc

NC = 2
NS = 16
NW = NC * NS
NB = 4
AH = 2
SCH = 64
LANES = 16


def _round_up(v, m):
  return (v + m - 1) // m * m


def _make_seg_sum(nacc, nquads, feat, tc_tiling=False):
  rpt = nacc // NS
  nz = rpt // SCH

  mesh = plsc.VectorSubcoreMesh(core_axis_name="c", subcore_axis_name="s",
                                num_cores=NC, num_subcores=NS)

  out_type = jax.ShapeDtypeStruct((NC * nacc, feat), jnp.float32)
  scratch = (
      [pltpu.VMEM((NB, SCH), jnp.int32) for _ in range(4)]
      + [pltpu.VMEM((SCH,), jnp.int32)]
      + [pltpu.VMEM((SCH, feat), jnp.float32) for _ in range(NB)]
      + [pltpu.VMEM_SHARED((nacc, feat), jnp.float32)]
      + [pltpu.SemaphoreType.DMA] * (2 * NB + 2)
  )

  def body(h_hbm, src_hbm, dst_hbm, out_hbm, *rest):
    qsrc0, qsrc1, qdst0, qdst1, dmy = rest[:5]
    bufs = rest[5:5 + NB]
    acc = rest[5 + NB]
    gsem = rest[6 + NB:6 + 2 * NB]
    ssem = rest[6 + 2 * NB:6 + 3 * NB]
    semsi, semdi = rest[6 + 3 * NB:]
    c = lax.axis_index("c")
    s = lax.axis_index("s")
    w = c * NS + s
    base = w * nquads
    tail = NW * nquads + w
    b0 = bufs[0]
    qsrc = (qsrc0, qsrc1)
    qdst = (qdst0, qdst1)

    zero16 = jnp.zeros((LANES,), jnp.float32)

    def zrow(i, carry):
      r = i // (feat // LANES)
      k = i % (feat // LANES)
      b0[r, pl.ds(k * LANES, LANES)] = zero16
      return carry
    lax.fori_loop(0, SCH * (feat // LANES), zrow, 0)
    r0 = s * rpt
    for j in range(nz):
      pltpu.sync_copy(b0, acc.at[pl.ds(r0 + j * SCH, SCH)])

    iota16 = lax.iota(jnp.int32, LANES)
    for k in range(SCH // LANES):
      dmy[pl.ds(k * LANES, LANES)] = iota16 + (nacc - SCH + k * LANES)

    pltpu.sync_copy(src_hbm.at[base], qsrc0)
    pltpu.sync_copy(dst_hbm.at[base], qdst0)

    plsc.subcore_barrier()

    for k in range(AH, NB):
      pltpu.async_copy(b0, acc.at[dmy], ssem[k], add=True)
    for k in range(AH):
      pltpu.async_copy(h_hbm.at[qsrc0.at[k]], bufs[k], gsem[k])

    def groupstep(gg, carry):
      for half in range(2):
        grp = gg * 2 + half
        sq_cur, dq_cur = qsrc[half], qdst[half]
        sq_nxt, dq_nxt = qsrc[1 - half], qdst[1 - half]
        ridx = jnp.where(grp + 1 == nquads, tail, base + grp + 1)
        for q in range(NB):
          if q == 0:
            pltpu.async_copy(src_hbm.at[ridx], sq_nxt, semsi)
            pltpu.async_copy(dst_hbm.at[ridx], dq_nxt, semdi)
          nb = (q + AH) % NB
          pltpu.make_async_copy(bufs[nb], acc.at[dmy], ssem[nb]).wait()
          if q == NB - AH:
            pltpu.make_async_copy(src_hbm.at[ridx], sq_nxt, semsi).wait()
            pltpu.make_async_copy(dst_hbm.at[ridx], dq_nxt, semdi).wait()
          nrow = (sq_cur.at[q + AH] if q < NB - AH
                  else sq_nxt.at[q - (NB - AH)])
          pltpu.async_copy(h_hbm.at[nrow], bufs[nb], gsem[nb])
          pltpu.make_async_copy(h_hbm.at[sq_cur.at[q]],
                                bufs[q], gsem[q]).wait()
          pltpu.async_copy(bufs[q], acc.at[dq_cur.at[q]],
                           ssem[q], add=True)
      return carry
    lax.fori_loop(0, nquads // 2, groupstep, 0)

    for k in range(AH):
      pltpu.make_async_copy(h_hbm.at[qsrc0.at[k]], bufs[k], gsem[k]).wait()
    for k in range(AH, NB):
      pltpu.make_async_copy(bufs[k], acc.at[dmy], ssem[k]).wait()

    plsc.subcore_barrier()

    pltpu.sync_copy(acc.at[pl.ds(r0, rpt)],
                    out_hbm.at[pl.ds(c * nacc + r0, rpt)])

  return pl.kernel(
      body,
      out_type=out_type,
      mesh=mesh,
      scratch_types=scratch,
      compiler_params=pltpu.CompilerParams(use_tc_tiling_on_sc=tc_tiling),
  )


def _lstm_body(xr, wl, bl, wa, outr):
  xb = xr[...]
  gates = jnp.dot(xb, wl[...], preferred_element_type=jnp.float32) + bl[...]
  hid = gates.shape[-1] // 4
  g = gates[:, 2 * hid:3 * hid]
  o = gates[:, 3 * hid:]
  c = jax.nn.sigmoid(gates[:, :hid]) * jnp.tanh(g)
  h = jax.nn.sigmoid(o) * jnp.tanh(c)
  sc = jnp.dot(h, wa[...], preferred_element_type=jnp.float32)
  sc = sc - jnp.max(sc, axis=-1, keepdims=True)
  e = jnp.exp(sc)
  attn = e / jnp.sum(e, axis=-1, keepdims=True)
  outr[...] = jnp.concatenate(
      [attn * h, jnp.ones((xb.shape[0], LANES), jnp.float32)], axis=1)


def _sage_body(a0r, a1r, lr, wl, bl, wr, outr):
  hid = wl.shape[0]
  a0 = a0r[...]
  a1 = a1r[...]
  hh = lr[:, :hid]
  agg = a0[:, :hid] + a1[:, :hid] + hh
  deg = a0[:, hid:hid + 1] + a1[:, hid:hid + 1] + 1.0
  pre = (jnp.dot(agg / deg, wl[...], preferred_element_type=jnp.float32)
         + bl[...]
         + jnp.dot(hh, wr[...], preferred_element_type=jnp.float32))
  outr[...] = jnp.maximum(pre, 0.0)


def _final_body(lr, h1r, b0r, b1r, a0r, a1r, wl, bl, wr,
                cw1, cb1, cw2, cb2, mw, mb, pw, pb, outr):
  hid = wl.shape[0]
  hh = h1r[...]
  agg = b0r[...] + b1r[...] + hh
  deg = a0r[:, hid:hid + 1] + a1r[:, hid:hid + 1] + 1.0
  pre = (jnp.dot(agg / deg, wl[...], preferred_element_type=jnp.float32)
         + bl[...]
         + jnp.dot(hh, wr[...], preferred_element_type=jnp.float32))
  h2 = jnp.maximum(pre, 0.0)
  x0 = jnp.concatenate([lr[:, :hid], hh, h2], axis=1)
  xc = x0
  for cw, cb in ((cw1, cb1), (cw2, cb2)):
    t = jnp.dot(xc, cw[...], preferred_element_type=jnp.float32)
    xc = x0 * t + cb[...] + xc
  deep = jnp.maximum(
      jnp.dot(x0, mw[...], preferred_element_type=jnp.float32) + mb[...], 0.0)
  fused = jnp.concatenate([xc, deep], axis=1)
  outr[...] = jnp.dot(fused, pw[...],
                      preferred_element_type=jnp.float32) + pb[...]


def _full(shape):
  return pl.BlockSpec(shape, lambda i: (0,) * len(shape))


def _rows(bl, ncols):
  return pl.BlockSpec((bl, ncols), lambda i: (i, 0))


def kernel(x, edge_index, W_lstm, b_lstm, W_att, W_l1, b_l1, W_r1,
           W_l2, b_l2, W_r2, cross_w1, cross_b1, cross_w2, cross_b2,
           mlp_W1, mlp_b1, pred_W, pred_b):
  n, feat = x.shape
  hid = W_att.shape[0]
  e = edge_index.shape[1]
  d = 3 * hid

  nacc = _round_up(n + 8, NS * 128)
  np_ = nacc
  br = 1024
  nblk = np_ // br
  cpw = _round_up(e, NW * SCH) // (NW * SCH)
  cpw = _round_up(cpw, 2 * NB)
  nquads = cpw // NB
  e_proc = NW * cpw * SCH

  src = edge_index[0]
  dst = edge_index[1]
  pe = e_proc - e
  fill = jnp.arange(pe, dtype=jnp.int32)
  src_p = jnp.concatenate([src, (fill * 37) % n])
  dst_p = jnp.concatenate([dst, n + fill % (nacc - n)])
  dfill = jnp.arange(NW * NB * SCH, dtype=jnp.int32)
  srcp = jnp.concatenate(
      [src_p.reshape(NW * nquads, NB, SCH),
       ((dfill * 31) % n).reshape(NW, NB, SCH)], axis=0)
  dstp = jnp.concatenate(
      [dst_p.reshape(NW * nquads, NB, SCH),
       (n + dfill % (nacc - n)).reshape(NW, NB, SCH)], axis=0)

  xp = jnp.pad(x, ((0, np_ - n), (0, 0)))
  bl2d = b_lstm.reshape(1, -1)
  b1 = b_l1.reshape(1, -1)
  b2 = b_l2.reshape(1, -1)
  cw1 = cross_w1.reshape(-1, 1)
  cb1 = cross_b1.reshape(1, -1)
  cw2 = cross_w2.reshape(-1, 1)
  cb2 = cross_b2.reshape(1, -1)
  mdim = mlp_W1.shape[1]
  mpad = 128 - mdim
  mw = jnp.pad(mlp_W1, ((0, 0), (0, mpad)))
  mb = jnp.pad(mlp_b1, (0, mpad)).reshape(1, -1)
  pw = jnp.concatenate(
      [pred_W[:d], pred_W[d:], jnp.zeros((mpad, 1), jnp.float32)], axis=0)
  pb = pred_b.reshape(1, 1)

  grid = (nblk,)
  faug = hid + LANES

  def _shift(k):
    return pl.BlockSpec((br, faug), lambda i, k=k: (i + k, 0))

  lstm = pl.pallas_call(
      _lstm_body,
      grid=grid,
      in_specs=[_rows(br, feat), _full((feat, 4 * hid)),
                _full((1, 4 * hid)), _full((hid, hid))],
      out_specs=_rows(br, faug),
      out_shape=jax.ShapeDtypeStruct((np_, faug), jnp.float32),
  )(xp, W_lstm, bl2d, W_att)

  seg_deg = _make_seg_sum(nacc, nquads, faug)
  agg_f = seg_deg(lstm, srcp, dstp)

  h1 = pl.pallas_call(
      _sage_body,
      grid=grid,
      in_specs=[_shift(0), _shift(nblk), _rows(br, faug),
                _full((hid, hid)), _full((1, hid)), _full((hid, hid))],
      out_specs=_rows(br, hid),
      out_shape=jax.ShapeDtypeStruct((np_, hid), jnp.float32),
  )(agg_f, agg_f, lstm, W_l1, b1, W_r1)

  seg2 = _make_seg_sum(nacc, nquads, hid)
  agg2_f = seg2(h1, srcp, dstp)

  def _shift2(k):
    return pl.BlockSpec((br, hid), lambda i, k=k: (i + k, 0))

  out = pl.pallas_call(
      _final_body,
      grid=grid,
      in_specs=[_rows(br, faug), _rows(br, hid),
                _shift2(0), _shift2(nblk),
                _shift(0), _shift(nblk),
                _full((hid, hid)), _full((1, hid)), _full((hid, hid)),
                _full((d, 1)), _full((1, d)), _full((d, 1)), _full((1, d)),
                _full((d, 128)), _full((1, 128)), _full((d + 128, 1)),
                _full((1, 1))],
      out_specs=_rows(br, 1),
      out_shape=jax.ShapeDtypeStruct((np_, 1), jnp.float32),
  )(lstm, h1, agg2_f, agg2_f, agg_f, agg_f, W_l2, b2, W_r2,
    cw1, cb1, cw2, cb2, mw, mb, pw, pb)

  return out[:n]
